# baseline (speedup 1.0000x reference)
import functools

import jax
import jax.numpy as jnp
from jax import lax
from jax.experimental import pallas as pl
from jax.experimental.pallas import tpu as pltpu
from jax.experimental.pallas import tpu_sc as plsc

D = 128
V = 4
NC = 2
NS = 16
NW = NC * NS
L = 16
BLK_R = 8
SUB = 400
NBUF = 2


def kernel(action_ids, emb_weight):
    B0, S = action_ids.shape
    B = B0 * S
    idx32 = action_ids.astype(jnp.int32)

    rows_w = B0 // NW
    n_blocks = rows_w // BLK_R
    n_sub = (BLK_R * S) // SUB
    n_chunks = n_blocks * n_sub
    assert rows_w * NW == B0 and n_blocks * BLK_R == rows_w
    assert n_sub * SUB == BLK_R * S and SUB % L == 0 and n_chunks % NBUF == 0
    assert SUB % S == 0

    mesh = plsc.VectorSubcoreMesh(core_axis_name="c", subcore_axis_name="s")

    @functools.partial(
        pl.kernel,
        mesh=mesh,
        out_type=jax.ShapeDtypeStruct((B, D), jnp.float32),
        compiler_params=pltpu.CompilerParams(needs_layout_passes=False),
        scratch_types=[
            pltpu.VMEM((V, D), jnp.float32),
            pltpu.VMEM((2, BLK_R, S), jnp.int32),
            pltpu.VMEM((NBUF * SUB, D), jnp.float32),
            pltpu.SemaphoreType.DMA,
            pltpu.SemaphoreType.DMA((2,)),
            pltpu.SemaphoreType.DMA((NBUF,)),
        ],
    )
    def sc_embed(idx_hbm, tab_hbm, out_hbm, tab_v, idx_v, rows_v,
                 sem_t, sem_i, sem_o):
        wid = lax.axis_index("s") * NC + lax.axis_index("c")
        rbase = wid * rows_w
        obase = wid * rows_w * S

        iota = lax.iota(jnp.int32, L)

        def start_idx(blk, s):
            pltpu.async_copy(
                idx_hbm.at[pl.ds(rbase + blk * BLK_R, BLK_R)],
                idx_v.at[s],
                sem_i.at[s],
            )

        def wait_idx(blk, s):
            pltpu.make_async_copy(
                idx_hbm.at[pl.ds(rbase + blk * BLK_R, BLK_R)],
                idx_v.at[s],
                sem_i.at[s],
            ).wait()

        def start_out(g, b):
            pltpu.async_copy(
                rows_v.at[pl.ds(b * SUB, SUB)],
                out_hbm.at[pl.ds(obase + g * SUB, SUB)],
                sem_o.at[b],
            )

        def wait_out(g, b):
            pltpu.make_async_copy(
                rows_v.at[pl.ds(b * SUB, SUB)],
                out_hbm.at[pl.ds(obase + g * SUB, SUB)],
                sem_o.at[b],
            ).wait()

        def expand(s, c, b):
            r0 = c * (SUB // S)
            sel_s = jnp.full((L,), s, jnp.int32)

            def grp(k, carry):
                p = k * L + iota
                rowv = r0 + p // S
                colv = p % S
                v16 = plsc.load_gather(idx_v, [sel_s, rowv, colv])
                row0 = b * SUB + k * L

                @plsc.parallel_loop(0, L, 1, unroll=4)
                def _(i):
                    sel = jnp.broadcast_to(i, (L,)).astype(jnp.int32)
                    splat_vi = jnp.take_along_axis(v16, sel, axis=0)
                    for jj in range(D // L):
                        col = iota + (jj * L)
                        vals = plsc.load_gather(tab_v, [splat_vi, col])
                        rows_v[row0 + i, pl.ds(jj * L, L)] = vals

                return carry
            lax.fori_loop(0, SUB // L, grp, None)

        pltpu.async_copy(tab_hbm, tab_v, sem_t).wait()
        for s in range(2):
            start_idx(s, s)

        def block(i2, carry):
            for s in range(2):
                blk = i2 * 2 + s
                wait_idx(blk, s)
                for c in range(n_sub):
                    g = blk * n_sub + c
                    b = c % NBUF

                    @pl.when(g >= NBUF)
                    def _():
                        wait_out(g - NBUF, b)

                    expand(s, c, b)
                    start_out(g, b)

                @pl.when(blk + 2 < n_blocks)
                def _():
                    start_idx(blk + 2, s)
            return carry

        lax.fori_loop(0, n_blocks // 2, block, None)

        for b0 in range(NBUF):
            g = n_chunks - NBUF + b0
            wait_out(g, g % NBUF)

    out = sc_embed(idx32, emb_weight)
    return out.reshape(B0, S, D)
